# baseline (speedup 1.0000x reference)
import functools

import jax
import jax.numpy as jnp
from jax import lax
from jax.experimental import pallas as pl
from jax.experimental.pallas import tpu as pltpu
from jax.experimental.pallas import tpu_sc as plsc

_DIM = 256
_NE = 8192
_TM = 256
_CK = 1024

_NC = 2
_NS = 16
_NW = _NC * _NS
_CH = 128


def _argmin_tile(x_ref, em2_ref, e2_ref, idx_ref, dsum_ref):
    x = x_ref[...]
    x2 = jnp.sum(x * x, axis=1, keepdims=True)
    best_s = None
    best_i = None
    _Q = 4
    _QW = _CK // _Q
    ii = lax.broadcasted_iota(jnp.int32, (_TM, _QW), 1).astype(jnp.float32)
    iis = [ii + float(q * _QW) for q in range(_Q)]
    for j in range(_NE // _CK):
        em2 = em2_ref[:, j * _CK:(j + 1) * _CK]
        e2 = e2_ref[:, j * _CK:(j + 1) * _CK]
        mm2 = jnp.dot(x, em2, preferred_element_type=jnp.float32)
        s = mm2 + e2
        m = jnp.min(s, axis=1, keepdims=True)
        packed = jnp.full((_TM, _QW), float(_NE), jnp.float32)
        for q in range(_Q - 1, -1, -1):
            sq = s[:, q * _QW:(q + 1) * _QW]
            packed = jnp.where(sq == m, iis[q], packed)
        cand = jnp.min(packed, axis=1, keepdims=True) + float(j * _CK)
        if best_s is None:
            best_s, best_i = m, cand
        else:
            better = m < best_s
            best_i = jnp.where(better, cand, best_i)
            best_s = jnp.where(better, m, best_s)
    idx_ref[...] = best_i.astype(jnp.int32)
    dsum_ref[...] = jnp.sum(x2 + best_s, axis=0, keepdims=True).reshape(1, 1, 1)


def _tc_argmin(flat, em2, e2):
    nt = flat.shape[0] // _TM
    idx, dsum = pl.pallas_call(
        _argmin_tile,
        grid=(nt,),
        in_specs=[
            pl.BlockSpec((_TM, _DIM), lambda i: (i, 0)),
            pl.BlockSpec((_DIM, _NE), lambda i: (0, 0)),
            pl.BlockSpec((1, _NE), lambda i: (0, 0)),
        ],
        out_specs=[
            pl.BlockSpec((_TM, 1), lambda i: (i, 0)),
            pl.BlockSpec((1, 1, 1), lambda i: (i, 0, 0)),
        ],
        out_shape=[
            jax.ShapeDtypeStruct((flat.shape[0], 1), jnp.int32),
            jax.ShapeDtypeStruct((nt, 1, 1), jnp.float32),
        ],
    )(flat, em2, e2)
    return idx[:, 0], dsum


def _sc_gather(table, idx):
    B = idx.shape[0]
    bw = B // _NW
    nch = bw // _CH
    idx3 = idx.reshape(_NW, nch, _CH)
    mesh = plsc.VectorSubcoreMesh(core_axis_name="c", subcore_axis_name="s")

    @functools.partial(
        pl.kernel,
        mesh=mesh,
        out_type=jax.ShapeDtypeStruct((B, _DIM), jnp.float32),
        scratch_types=[
            pltpu.VMEM((nch, _CH), jnp.int32),
            pltpu.VMEM((_CH, _DIM), jnp.float32),
            pltpu.VMEM((_CH, _DIM), jnp.float32),
            pltpu.VMEM((_CH, _DIM), jnp.float32),
            pltpu.SemaphoreType.DMA,
            pltpu.SemaphoreType.DMA,
            pltpu.SemaphoreType.DMA,
            pltpu.SemaphoreType.DMA,
            pltpu.SemaphoreType.DMA,
            pltpu.SemaphoreType.DMA,
        ],
    )
    def k(table_hbm, idx_hbm, out_hbm, idx_v, b0, b1, b2, g0, g1, g2, s0, s1, s2):
        wid = lax.axis_index("s") * _NC + lax.axis_index("c")
        base = wid * bw
        bufs = (b0, b1, b2)
        gsem = (g0, g1, g2)
        ssem = (s0, s1, s2)

        def gather(c):
            return pltpu.make_async_copy(
                table_hbm.at[idx_v.at[c]], bufs[c % 3], gsem[c % 3])

        def store(c):
            return pltpu.make_async_copy(
                bufs[c % 3], out_hbm.at[pl.ds(base + c * _CH, _CH)], ssem[c % 3])

        pltpu.sync_copy(idx_hbm.at[wid], idx_v)
        for c in range(min(3, nch)):
            gather(c).start()
        for c in range(nch):
            gather(c).wait()
            store(c).start()
            if c + 3 < nch:
                store(c).wait()
                gather(c + 3).start()
        for c in range(max(0, nch - 3), nch):
            store(c).wait()

    return k(table, idx3)


def kernel(inputs, embed):
    flat = inputs.reshape(-1, _DIM)
    e2 = jnp.sum(embed ** 2, axis=0, keepdims=True)
    em2 = -2.0 * embed
    et = embed.T
    n = flat.shape[0]
    nsl = 4
    sl = n // nsl
    idxs, dsums, qs = [], [], []
    for i in range(nsl):
        idx_i, dsum_i = _tc_argmin(flat[i * sl:(i + 1) * sl], em2, e2)
        idxs.append(idx_i)
        dsums.append(dsum_i)
        qs.append(_sc_gather(et, idx_i))
    idx = jnp.concatenate(idxs)
    q = jnp.concatenate(qs)
    quantize = q.reshape(inputs.shape)
    diff = sum(jnp.sum(d) for d in dsums) / (n * _DIM)
    embed_ind = idx.reshape(inputs.shape[:-1])
    return (quantize, diff, embed_ind)


# speedup vs baseline: 1.0501x; 1.0501x over previous
import functools

import jax
import jax.numpy as jnp
from jax import lax
from jax.experimental import pallas as pl
from jax.experimental.pallas import tpu as pltpu
from jax.experimental.pallas import tpu_sc as plsc

_DIM = 256
_NE = 8192
_TM = 256
_CK = 512

_NC = 2
_NS = 16
_NW = _NC * _NS
_CH = 128


def _argmin_tile(x_ref, em2_ref, e2_ref, idx_ref, dsum_ref):
    x = x_ref[...]
    x2 = jnp.sum(x * x, axis=1, keepdims=True)
    best_s = None
    best_i = None
    _Q = 2
    _QW = _CK // _Q
    ii = lax.broadcasted_iota(jnp.int32, (_TM, _QW), 1).astype(jnp.float32)
    iis = [ii + float(q * _QW) for q in range(_Q)]
    for j in range(_NE // _CK):
        em2 = em2_ref[:, j * _CK:(j + 1) * _CK]
        e2 = e2_ref[:, j * _CK:(j + 1) * _CK]
        mm2 = jnp.dot(x, em2, preferred_element_type=jnp.float32)
        s = mm2 + e2
        m = jnp.min(s, axis=1, keepdims=True)
        packed = jnp.full((_TM, _QW), float(_NE), jnp.float32)
        for q in range(_Q - 1, -1, -1):
            sq = s[:, q * _QW:(q + 1) * _QW]
            packed = jnp.where(sq == m, iis[q], packed)
        cand = jnp.min(packed, axis=1, keepdims=True) + float(j * _CK)
        if best_s is None:
            best_s, best_i = m, cand
        else:
            better = m < best_s
            best_i = jnp.where(better, cand, best_i)
            best_s = jnp.where(better, m, best_s)
    idx_ref[...] = best_i.astype(jnp.int32)
    dsum_ref[...] = jnp.sum(x2 + best_s, axis=0, keepdims=True).reshape(1, 1, 1)


def _tc_argmin(flat, em2, e2):
    nt = flat.shape[0] // _TM
    idx, dsum = pl.pallas_call(
        _argmin_tile,
        grid=(nt,),
        in_specs=[
            pl.BlockSpec((_TM, _DIM), lambda i: (i, 0)),
            pl.BlockSpec((_DIM, _NE), lambda i: (0, 0)),
            pl.BlockSpec((1, _NE), lambda i: (0, 0)),
        ],
        out_specs=[
            pl.BlockSpec((_TM, 1), lambda i: (i, 0)),
            pl.BlockSpec((1, 1, 1), lambda i: (i, 0, 0)),
        ],
        out_shape=[
            jax.ShapeDtypeStruct((flat.shape[0], 1), jnp.int32),
            jax.ShapeDtypeStruct((nt, 1, 1), jnp.float32),
        ],
    )(flat, em2, e2)
    return idx[:, 0], dsum


def _sc_gather(table, idx):
    B = idx.shape[0]
    bw = B // _NW
    nch = bw // _CH
    idx3 = idx.reshape(_NW, nch, _CH)
    mesh = plsc.VectorSubcoreMesh(core_axis_name="c", subcore_axis_name="s")

    @functools.partial(
        pl.kernel,
        mesh=mesh,
        out_type=jax.ShapeDtypeStruct((B, _DIM), jnp.float32),
        scratch_types=[
            pltpu.VMEM((nch, _CH), jnp.int32),
            pltpu.VMEM((_CH, _DIM), jnp.float32),
            pltpu.VMEM((_CH, _DIM), jnp.float32),
            pltpu.VMEM((_CH, _DIM), jnp.float32),
            pltpu.SemaphoreType.DMA,
            pltpu.SemaphoreType.DMA,
            pltpu.SemaphoreType.DMA,
            pltpu.SemaphoreType.DMA,
            pltpu.SemaphoreType.DMA,
            pltpu.SemaphoreType.DMA,
        ],
    )
    def k(table_hbm, idx_hbm, out_hbm, idx_v, b0, b1, b2, g0, g1, g2, s0, s1, s2):
        wid = lax.axis_index("s") * _NC + lax.axis_index("c")
        base = wid * bw
        bufs = (b0, b1, b2)
        gsem = (g0, g1, g2)
        ssem = (s0, s1, s2)

        def gather(c):
            return pltpu.make_async_copy(
                table_hbm.at[idx_v.at[c]], bufs[c % 3], gsem[c % 3])

        def store(c):
            return pltpu.make_async_copy(
                bufs[c % 3], out_hbm.at[pl.ds(base + c * _CH, _CH)], ssem[c % 3])

        pltpu.sync_copy(idx_hbm.at[wid], idx_v)
        for c in range(min(3, nch)):
            gather(c).start()
        for c in range(nch):
            gather(c).wait()
            store(c).start()
            if c + 3 < nch:
                store(c).wait()
                gather(c + 3).start()
        for c in range(max(0, nch - 3), nch):
            store(c).wait()

    return k(table, idx3)


def kernel(inputs, embed):
    flat = inputs.reshape(-1, _DIM)
    e2 = jnp.sum(embed ** 2, axis=0, keepdims=True)
    em2 = -2.0 * embed
    et = embed.T
    idx, dsum = _tc_argmin(flat, em2, e2)
    q = _sc_gather(et, idx)
    quantize = q.reshape(inputs.shape)
    diff = jnp.sum(dsum) / (flat.shape[0] * _DIM)
    embed_ind = idx.reshape(inputs.shape[:-1])
    return (quantize, diff, embed_ind)


# speedup vs baseline: 1.0730x; 1.0218x over previous
import functools

import jax
import jax.numpy as jnp
from jax import lax
from jax.experimental import pallas as pl
from jax.experimental.pallas import tpu as pltpu
from jax.experimental.pallas import tpu_sc as plsc

_DIM = 256
_NE = 8192
_TM = 256
_CK = 1024

_NC = 2
_NS = 16
_NW = _NC * _NS
_CH = 128


def _argmin_tile(x_ref, em2_ref, e2_ref, idx_ref, dsum_ref):
    x = x_ref[...]
    x2 = jnp.sum(x * x, axis=1, keepdims=True)
    best_s = None
    best_i = None
    _Q = 4
    _QW = _CK // _Q
    ii = lax.broadcasted_iota(jnp.int32, (_TM, _QW), 1).astype(jnp.float32)
    iis = [ii + float(q * _QW) for q in range(_Q)]
    for j in range(_NE // _CK):
        em2 = em2_ref[:, j * _CK:(j + 1) * _CK]
        e2 = e2_ref[:, j * _CK:(j + 1) * _CK]
        mm2 = jnp.dot(x, em2, preferred_element_type=jnp.float32)
        s = mm2 + e2
        m = jnp.min(s, axis=1, keepdims=True)
        packed = jnp.full((_TM, _QW), float(_NE), jnp.float32)
        for q in range(_Q - 1, -1, -1):
            sq = s[:, q * _QW:(q + 1) * _QW]
            packed = jnp.where(sq == m, iis[q], packed)
        cand = jnp.min(packed, axis=1, keepdims=True) + float(j * _CK)
        if best_s is None:
            best_s, best_i = m, cand
        else:
            better = m < best_s
            best_i = jnp.where(better, cand, best_i)
            best_s = jnp.where(better, m, best_s)
    idx_ref[...] = best_i.astype(jnp.int32)
    dsum_ref[...] = jnp.sum(x2 + best_s, axis=0, keepdims=True).reshape(1, 1, 1)


def _tc_argmin(flat, em2, e2):
    nt = flat.shape[0] // _TM
    idx, dsum = pl.pallas_call(
        _argmin_tile,
        grid=(nt,),
        in_specs=[
            pl.BlockSpec((_TM, _DIM), lambda i: (i, 0)),
            pl.BlockSpec((_DIM, _NE), lambda i: (0, 0)),
            pl.BlockSpec((1, _NE), lambda i: (0, 0)),
        ],
        out_specs=[
            pl.BlockSpec((_TM, 1), lambda i: (i, 0)),
            pl.BlockSpec((1, 1, 1), lambda i: (i, 0, 0)),
        ],
        out_shape=[
            jax.ShapeDtypeStruct((flat.shape[0], 1), jnp.int32),
            jax.ShapeDtypeStruct((nt, 1, 1), jnp.float32),
        ],
    )(flat, em2, e2)
    return idx[:, 0], dsum


def _sc_gather(table, idx):
    B = idx.shape[0]
    bw = B // _NW
    nch = bw // _CH
    idx3 = idx.reshape(_NW, nch, _CH)
    mesh = plsc.VectorSubcoreMesh(core_axis_name="c", subcore_axis_name="s")

    @functools.partial(
        pl.kernel,
        mesh=mesh,
        out_type=jax.ShapeDtypeStruct((B, _DIM), jnp.float32),
        scratch_types=[
            pltpu.VMEM((nch, _CH), jnp.int32),
            pltpu.VMEM((_CH, _DIM), jnp.float32),
            pltpu.VMEM((_CH, _DIM), jnp.float32),
            pltpu.VMEM((_CH, _DIM), jnp.float32),
            pltpu.SemaphoreType.DMA,
            pltpu.SemaphoreType.DMA,
            pltpu.SemaphoreType.DMA,
            pltpu.SemaphoreType.DMA,
            pltpu.SemaphoreType.DMA,
            pltpu.SemaphoreType.DMA,
        ],
    )
    def k(table_hbm, idx_hbm, out_hbm, idx_v, b0, b1, b2, g0, g1, g2, s0, s1, s2):
        wid = lax.axis_index("s") * _NC + lax.axis_index("c")
        base = wid * bw
        bufs = (b0, b1, b2)
        gsem = (g0, g1, g2)
        ssem = (s0, s1, s2)

        def gather(c):
            return pltpu.make_async_copy(
                table_hbm.at[idx_v.at[c]], bufs[c % 3], gsem[c % 3])

        def store(c):
            return pltpu.make_async_copy(
                bufs[c % 3], out_hbm.at[pl.ds(base + c * _CH, _CH)], ssem[c % 3])

        pltpu.sync_copy(idx_hbm.at[wid], idx_v)
        for c in range(min(3, nch)):
            gather(c).start()
        for c in range(nch):
            gather(c).wait()
            store(c).start()
            if c + 3 < nch:
                store(c).wait()
                gather(c + 3).start()
        for c in range(max(0, nch - 3), nch):
            store(c).wait()

    return k(table, idx3)


def kernel(inputs, embed):
    flat = inputs.reshape(-1, _DIM)
    e2 = jnp.sum(embed ** 2, axis=0, keepdims=True)
    em2 = -2.0 * embed
    et = embed.T
    idx, dsum = _tc_argmin(flat, em2, e2)
    q = _sc_gather(et, idx)
    quantize = q.reshape(inputs.shape)
    diff = jnp.sum(dsum) / (flat.shape[0] * _DIM)
    embed_ind = idx.reshape(inputs.shape[:-1])
    return (quantize, diff, embed_ind)
